# baseline (speedup 1.0000x reference)
import functools

import jax
import jax.numpy as jnp
from jax import lax
from jax.experimental import pallas as pl
from jax.experimental.pallas import tpu as pltpu
from jax.experimental.pallas import tpu_sc as plsc

_NC = 2
_NS = 16
_NW = _NC * _NS
_L = 16

_DIM = 128
_VOCAB = 10

_UNIT = 4096


@functools.lru_cache(maxsize=None)
def _build_sc_gather(n_rows: int, n_cols: int):
    q_per_line = n_rows // _UNIT
    n_units = n_cols * q_per_line
    assert n_rows % _UNIT == 0 and n_units % _NW == 0
    u_per_w = n_units // _NW

    mesh = plsc.VectorSubcoreMesh(core_axis_name="c", subcore_axis_name="s")

    @functools.partial(
        pl.kernel,
        mesh=mesh,
        out_type=jax.ShapeDtypeStruct((n_cols, 1, n_rows), jnp.float32),
        scratch_types=[
            pltpu.VMEM((_VOCAB * _DIM + _DIM + 8,), jnp.float32),
            pltpu.VMEM((_L,), jnp.float32),
            pltpu.VMEM((_UNIT,), jnp.int32),
            pltpu.VMEM((_UNIT,), jnp.int32),
            pltpu.VMEM((_UNIT,), jnp.int32),
            pltpu.VMEM((_UNIT,), jnp.float32),
            pltpu.VMEM((_UNIT,), jnp.float32),
            pltpu.SemaphoreType.DMA,
            pltpu.SemaphoreType.DMA,
            pltpu.SemaphoreType.DMA,
            pltpu.SemaphoreType.DMA,
            pltpu.SemaphoreType.DMA,
        ],
        compiler_params=pltpu.CompilerParams(
            needs_layout_passes=False, use_tc_tiling_on_sc=True,
            disable_bounds_checks=True),
    )
    def sc_kernel(idx_hbm, aux_hbm, out_hbm,
                  aux_v, val_tab_v,
                  idx_v0, idx_v1, idx_v2, out_v0, out_v1,
                  in_sem0, in_sem1, in_sem2, out_sem0, out_sem1):
        wid = lax.axis_index("s") * _NC + lax.axis_index("c")
        u0 = wid * u_per_w

        idx_bufs = [idx_v0, idx_v1, idx_v2]
        out_bufs = [out_v0, out_v1]
        in_sems = [in_sem0, in_sem1, in_sem2]
        out_sems = [out_sem0, out_sem1]

        def start_in(i):
            u = u0 + i
            line = u // q_per_line
            r_off = (u % q_per_line) * _UNIT
            return pltpu.async_copy(
                idx_hbm.at[line, pl.ds(r_off, _UNIT)], idx_bufs[i % 3],
                in_sems[i % 3])

        in_copies = [None] * u_per_w
        out_copies = [None] * u_per_w
        in_copies[0] = start_in(0)
        in_copies[1] = start_in(1)

        pltpu.sync_copy(aux_hbm, aux_v)

        lane = lax.iota(jnp.int32, _L)
        col_idx = jnp.where(lane < _VOCAB, lane, 0) * _DIM
        one = jnp.ones((_L,), jnp.int32)
        v_vec = plsc.load_gather(
            aux_v, [jnp.full((_L,), _VOCAB * _DIM + _DIM, jnp.int32)])
        for dg in range(_DIM // _L):
            w_vec = aux_v[pl.ds(_VOCAB * _DIM + dg * _L, _L)]
            for j in range(_L):
                col = plsc.load_gather(aux_v, [col_idx])
                col_idx = col_idx + one
                v_vec = v_vec + col * w_vec[j]
        val_tab_v[...] = v_vec

        for i in range(u_per_w):
            if i + 2 < u_per_w:
                in_copies[i + 2] = start_in(i + 2)
            in_copies[i].wait()
            if i >= 2:
                out_copies[i - 2].wait()

            idx_b = idx_bufs[i % 3]
            out_b = out_bufs[i % 2]

            @plsc.parallel_loop(0, _UNIT, step=_L, unroll=8)
            def body(j, idx_b=idx_b, out_b=out_b):
                out_b[pl.ds(j, _L)] = plsc.load_gather(
                    val_tab_v, [idx_b[pl.ds(j, _L)]])

            u = u0 + i
            out_copies[i] = pltpu.async_copy(
                out_b,
                out_hbm.at[u // q_per_line, 0,
                           pl.ds((u % q_per_line) * _UNIT, _UNIT)],
                out_sems[i % 2])
        out_copies[u_per_w - 2].wait()
        out_copies[u_per_w - 1].wait()

    return sc_kernel


def kernel(input, emb_table, lin_w, lin_b):
    bsz, seq = input.shape
    idx_t = input.astype(jnp.int32).T
    aux = jnp.concatenate([
        emb_table.astype(jnp.float32).reshape(_VOCAB * _DIM),
        lin_w.astype(jnp.float32).reshape(_DIM),
        lin_b.astype(jnp.float32).reshape(1),
        jnp.zeros((7,), jnp.float32),
    ])
    cube = _build_sc_gather(bsz, seq)(idx_t, aux)
    return cube.transpose(2, 0, 1)


# speedup vs baseline: 1.0479x; 1.0479x over previous
import functools

import jax
import jax.numpy as jnp
from jax import lax
from jax.experimental import pallas as pl
from jax.experimental.pallas import tpu as pltpu
from jax.experimental.pallas import tpu_sc as plsc

_NC = 2
_NS = 16
_NW = _NC * _NS
_L = 16

_DIM = 128
_VOCAB = 10

_UNIT = 4096


@functools.lru_cache(maxsize=None)
def _build_sc_gather(n_rows: int, n_cols: int):
    q_per_line = n_rows // _UNIT
    n_units = n_cols * q_per_line
    assert n_rows % _UNIT == 0 and n_units % _NW == 0
    u_per_w = n_units // _NW

    mesh = plsc.VectorSubcoreMesh(core_axis_name="c", subcore_axis_name="s")

    @functools.partial(
        pl.kernel,
        mesh=mesh,
        out_type=jax.ShapeDtypeStruct((n_cols, 1, n_rows), jnp.float32),
        scratch_types=[
            pltpu.VMEM((_DIM * _L,), jnp.float32),
            pltpu.VMEM((_DIM,), jnp.float32),
            pltpu.VMEM((_L,), jnp.float32),
            pltpu.VMEM((_L,), jnp.float32),
            pltpu.VMEM((_UNIT,), jnp.int32),
            pltpu.VMEM((_UNIT,), jnp.int32),
            pltpu.VMEM((_UNIT,), jnp.int32),
            pltpu.VMEM((_UNIT,), jnp.float32),
            pltpu.VMEM((_UNIT,), jnp.float32),
            pltpu.SemaphoreType.DMA,
            pltpu.SemaphoreType.DMA,
            pltpu.SemaphoreType.DMA,
            pltpu.SemaphoreType.DMA,
            pltpu.SemaphoreType.DMA,
        ],
        compiler_params=pltpu.CompilerParams(
            needs_layout_passes=False, use_tc_tiling_on_sc=True,
            disable_bounds_checks=True),
    )
    def sc_kernel(idx_hbm, tabt_hbm, w_hbm, b_hbm, out_hbm,
                  tabt_v, w_v, b_v, val_tab_v,
                  idx_v0, idx_v1, idx_v2, out_v0, out_v1,
                  in_sem0, in_sem1, in_sem2, out_sem0, out_sem1):
        wid = lax.axis_index("s") * _NC + lax.axis_index("c")
        u0 = wid * u_per_w

        idx_bufs = [idx_v0, idx_v1, idx_v2]
        out_bufs = [out_v0, out_v1]
        in_sems = [in_sem0, in_sem1, in_sem2]
        out_sems = [out_sem0, out_sem1]

        def start_in(i):
            u = u0 + i
            line = u // q_per_line
            r_off = (u % q_per_line) * _UNIT
            return pltpu.async_copy(
                idx_hbm.at[line, pl.ds(r_off, _UNIT)], idx_bufs[i % 3],
                in_sems[i % 3])

        in_copies = [None] * u_per_w
        out_copies = [None] * u_per_w
        in_copies[0] = start_in(0)
        in_copies[1] = start_in(1)

        pltpu.sync_copy(tabt_hbm, tabt_v)
        pltpu.sync_copy(w_hbm, w_v)
        pltpu.sync_copy(b_hbm, b_v)

        v_vec = b_v[...]
        for dg in range(_DIM // _L):
            w_vec = w_v[pl.ds(dg * _L, _L)]
            for j in range(_L):
                d = dg * _L + j
                v_vec = v_vec + tabt_v[pl.ds(d * _L, _L)] * w_vec[j]
        val_tab_v[...] = v_vec

        for i in range(u_per_w):
            if i + 2 < u_per_w:
                in_copies[i + 2] = start_in(i + 2)
            in_copies[i].wait()
            if i >= 2:
                out_copies[i - 2].wait()

            idx_b = idx_bufs[i % 3]
            out_b = out_bufs[i % 2]

            @plsc.parallel_loop(0, _UNIT, step=_L, unroll=8)
            def body(j, idx_b=idx_b, out_b=out_b):
                out_b[pl.ds(j, _L)] = plsc.load_gather(
                    val_tab_v, [idx_b[pl.ds(j, _L)]])

            u = u0 + i
            out_copies[i] = pltpu.async_copy(
                out_b,
                out_hbm.at[u // q_per_line, 0,
                           pl.ds((u % q_per_line) * _UNIT, _UNIT)],
                out_sems[i % 2])
        out_copies[u_per_w - 2].wait()
        out_copies[u_per_w - 1].wait()

    return sc_kernel


def kernel(input, emb_table, lin_w, lin_b):
    bsz, seq = input.shape
    idx_t = input.astype(jnp.int32).T
    tabt = jnp.pad(emb_table.astype(jnp.float32).T,
                   ((0, 0), (0, _L - _VOCAB))).reshape(_DIM * _L)
    w = lin_w.astype(jnp.float32).reshape(_DIM)
    b = jnp.broadcast_to(lin_b.astype(jnp.float32).reshape(1), (_L,))
    cube = _build_sc_gather(bsz, seq)(idx_t, tabt, w, b)
    return cube.transpose(2, 0, 1)
